# Initial kernel scaffold; baseline (speedup 1.0000x reference)
import jax
import jax.numpy as jnp
from jax.experimental import pallas as pl


def kernel(node_embeddings, edge_index, edge_encodings, W1, b1, W2, b2):
    raise NotImplementedError("write your pallas kernel here")


# speedup vs baseline: 4.7982x; 4.7982x over previous
import functools

import jax
import jax.numpy as jnp
from jax import lax
from jax.experimental import pallas as pl
from jax.experimental.pallas import tpu as pltpu
from jax.experimental.pallas import tpu_sc as plsc

N = 10000
NP = 10240
E = 320000
D1 = 128
D2 = 64
NC = 2
NS = 16
EPC = E // NC
EPT = EPC // NS
K = 80
CHUNKS = EPT // K
RPT = NP // NS

_MESH = plsc.VectorSubcoreMesh(core_axis_name="c", subcore_axis_name="s")


def _zero_vec_f32(ref, n):
    def body(i, carry):
        ref[pl.ds(i * 16, 16)] = jnp.zeros((16,), jnp.float32)
        return carry
    lax.fori_loop(0, n // 16, body, 0)


@functools.partial(
    pl.kernel,
    mesh=_MESH,
    out_type=jax.ShapeDtypeStruct((NC, 2, NP), jnp.float32),
    scratch_types=[
        pltpu.VMEM((K,), jnp.int32),
        pltpu.VMEM((K,), jnp.float32),
        pltpu.VMEM((RPT,), jnp.float32),
        pltpu.VMEM_SHARED((NP,), jnp.float32),
        pltpu.VMEM_SHARED((NP,), jnp.float32),
    ],
)
def _sc_degrees(src_hbm, dst_hbm, out_hbm, idx_v, ones_v, zeros_v, dego, degi):
    c = lax.axis_index("c")
    s = lax.axis_index("s")

    def fill_ones(i, carry):
        ones_v[pl.ds(i * 16, 16)] = jnp.ones((16,), jnp.float32)
        return carry
    lax.fori_loop(0, K // 16, fill_ones, 0)
    _zero_vec_f32(zeros_v, RPT)

    row0 = s * RPT
    pltpu.sync_copy(zeros_v, dego.at[pl.ds(row0, RPT)])
    pltpu.sync_copy(zeros_v, degi.at[pl.ds(row0, RPT)])
    plsc.subcore_barrier()

    base = c * EPC + s * EPT

    def step(i, carry):
        off = pl.multiple_of(base + i * K, 8)
        pltpu.sync_copy(src_hbm.at[pl.ds(off, K)], idx_v)
        pltpu.sync_copy(ones_v, dego.at[idx_v], add=True)
        pltpu.sync_copy(dst_hbm.at[pl.ds(off, K)], idx_v)
        pltpu.sync_copy(ones_v, degi.at[idx_v], add=True)
        return carry
    lax.fori_loop(0, CHUNKS, step, 0)
    plsc.subcore_barrier()

    pltpu.sync_copy(dego.at[pl.ds(row0, RPT)], out_hbm.at[c, 0, pl.ds(row0, RPT)])
    pltpu.sync_copy(degi.at[pl.ds(row0, RPT)], out_hbm.at[c, 1, pl.ds(row0, RPT)])


def _make_segsum(d):
    @functools.partial(
        pl.kernel,
        mesh=_MESH,
        compiler_params=pltpu.CompilerParams(use_tc_tiling_on_sc=False),
        out_type=jax.ShapeDtypeStruct((NC, NP, d), jnp.float32),
        scratch_types=[
            pltpu.VMEM((K,), jnp.int32),
            pltpu.VMEM((K,), jnp.int32),
            pltpu.VMEM((K, d), jnp.float32),
            pltpu.VMEM_SHARED((NP, d), jnp.float32),
            pltpu.SemaphoreType.DMA,
        ],
    )
    def seg(z_hbm, src_hbm, dst_hbm, out_hbm, sidx, didx, gbuf, accum, sem):
        c = lax.axis_index("c")
        s = lax.axis_index("s")

        def zrow(i, carry):
            r = i // (d // 16)
            col = (i % (d // 16)) * 16
            gbuf[r, pl.ds(col, 16)] = jnp.zeros((16,), jnp.float32)
            return carry
        lax.fori_loop(0, K * d // 16, zrow, 0)
        for j in range(RPT // K):
            pltpu.sync_copy(gbuf, accum.at[pl.ds(s * RPT + j * K, K)])
        plsc.subcore_barrier()

        base = c * EPC + s * EPT

        def step(i, carry):
            off = pl.multiple_of(base + i * K, 8)
            pltpu.sync_copy(src_hbm.at[pl.ds(off, K)], sidx)
            pltpu.sync_copy(dst_hbm.at[pl.ds(off, K)], didx)
            pltpu.async_copy(z_hbm.at[sidx], gbuf, sem).wait()
            pltpu.sync_copy(gbuf, accum.at[didx], add=True)
            return carry
        lax.fori_loop(0, CHUNKS, step, 0)
        plsc.subcore_barrier()

        for j in range(RPT // K):
            r = s * RPT + j * K
            pltpu.sync_copy(accum.at[pl.ds(r, K)], out_hbm.at[c, pl.ds(r, K)])

    return seg


_sc_segsum_128 = _make_segsum(D1)
_sc_segsum_64 = _make_segsum(D2)


_BN = 1024


def _norms(deg_ref, which):
    deg = deg_ref[0, which, :] + deg_ref[1, which, :]
    return lax.rsqrt(jnp.maximum(deg, 1.0))


def _tc1_body(x_ref, w_ref, deg_ref, o_ref):
    ns = _norms(deg_ref, 0)
    z = jnp.dot(x_ref[...], w_ref[...], preferred_element_type=jnp.float32)
    o_ref[...] = z * ns[:, None]


def _tc1(x, w1, degp):
    return pl.pallas_call(
        _tc1_body,
        grid=(NP // _BN,),
        in_specs=[
            pl.BlockSpec((_BN, D1), lambda i: (i, 0)),
            pl.BlockSpec((D1, D1), lambda i: (0, 0)),
            pl.BlockSpec((NC, 2, _BN), lambda i: (0, 0, i)),
        ],
        out_specs=pl.BlockSpec((_BN, D1), lambda i: (i, 0)),
        out_shape=jax.ShapeDtypeStruct((NP, D1), jnp.float32),
    )(x, w1, degp)


def _tc2_body(a_ref, deg_ref, b1_ref, w_ref, o_ref):
    a = a_ref[0] + a_ref[1]
    nd = _norms(deg_ref, 1)
    ns = _norms(deg_ref, 0)
    h = jnp.maximum(a * nd[:, None] + b1_ref[...][None, :], 0.0)
    z = jnp.dot(h, w_ref[...], preferred_element_type=jnp.float32)
    o_ref[...] = z * ns[:, None]


def _tc2(a1, degp, b1, w2):
    return pl.pallas_call(
        _tc2_body,
        grid=(NP // _BN,),
        in_specs=[
            pl.BlockSpec((NC, _BN, D1), lambda i: (0, i, 0)),
            pl.BlockSpec((NC, 2, _BN), lambda i: (0, 0, i)),
            pl.BlockSpec((D1,), lambda i: (0,)),
            pl.BlockSpec((D1, D2), lambda i: (0, 0)),
        ],
        out_specs=pl.BlockSpec((_BN, D2), lambda i: (i, 0)),
        out_shape=jax.ShapeDtypeStruct((NP, D2), jnp.float32),
    )(a1, degp, b1, w2)


def _tc3_body(a_ref, deg_ref, b2_ref, o_ref):
    a = a_ref[0] + a_ref[1]
    nd = _norms(deg_ref, 1)
    o_ref[...] = a * nd[:, None] + b2_ref[...][None, :]


def _tc3(a2, degp, b2):
    return pl.pallas_call(
        _tc3_body,
        grid=(NP // _BN,),
        in_specs=[
            pl.BlockSpec((NC, _BN, D2), lambda i: (0, i, 0)),
            pl.BlockSpec((NC, 2, _BN), lambda i: (0, 0, i)),
            pl.BlockSpec((D2,), lambda i: (0,)),
        ],
        out_specs=pl.BlockSpec((_BN, D2), lambda i: (i, 0)),
        out_shape=jax.ShapeDtypeStruct((NP, D2), jnp.float32),
    )(a2, degp, b2)


def kernel(node_embeddings, edge_index, edge_encodings, W1, b1, W2, b2):
    del edge_encodings
    src = edge_index[0]
    dst = edge_index[1]
    x = jnp.pad(node_embeddings, ((0, NP - N), (0, 0)))
    degp = _sc_degrees(src, dst)
    z1 = _tc1(x, W1, degp)
    a1 = _sc_segsum_128(z1, src, dst)
    z2 = _tc2(a1, degp, b1, W2)
    a2 = _sc_segsum_64(z2, src, dst)
    out = _tc3(a2, degp, b2)
    return out[:N]
